# baseline (speedup 1.0000x reference)
import functools

import jax
import jax.numpy as jnp
from jax import lax
from jax.experimental import pallas as pl
from jax.experimental.pallas import tpu as pltpu
from jax.experimental.pallas import tpu_sc as plsc

_ROWS = 8192
_DIM = 1024
_SC_ROWS = 4096
_NUM_WORKERS = 32
_CHUNK = 16
_NUM_CHUNKS = _SC_ROWS // (_NUM_WORKERS * _CHUNK)
_NBUF = 6
_DELAY = 3

_TC_BLOCK = 512

_MESH = plsc.VectorSubcoreMesh(core_axis_name="c", subcore_axis_name="s")


@functools.partial(
    pl.kernel,
    mesh=_MESH,
    out_type=jax.ShapeDtypeStruct((_ROWS, _DIM), jnp.float32),
    scratch_types=(
        [pltpu.VMEM((_CHUNK, _DIM), jnp.float32) for _ in range(_NBUF)]
        + [pltpu.SemaphoreType.DMA for _ in range(2 * _NBUF)]
    ),
)
def _gather_rows_sc(table_hbm, out_hbm, *scratch):
    bufs = scratch[:_NBUF]
    lsems = scratch[_NBUF : 2 * _NBUF]
    ssems = scratch[2 * _NBUF :]

    wid = lax.axis_index("s") * 2 + lax.axis_index("c")

    loads = [None] * _NBUF
    stores = [None] * _NBUF

    def _row0(j):
        return (j * _NUM_WORKERS + wid) * _CHUNK

    def _load(j):
        b = j % _NBUF
        loads[b] = pltpu.async_copy(
            table_hbm.at[pl.ds(_row0(j), _CHUNK)], bufs[b], lsems[b]
        )

    for j in range(min(_DELAY + 1, _NUM_CHUNKS)):
        _load(j)
    for i in range(_NUM_CHUNKS):
        b = i % _NBUF
        loads[b].wait()
        stores[b] = pltpu.async_copy(
            bufs[b], out_hbm.at[pl.ds(_row0(i), _CHUNK)], ssems[b]
        )
        j = i + _DELAY
        if _DELAY < j < _NUM_CHUNKS:
            bb = j % _NBUF
            if stores[bb] is not None:
                stores[bb].wait()
                stores[bb] = None
            _load(j)
    for st in stores:
        if st is not None:
            st.wait()


def _copy_rows_tc(weights, sc_out):
    base = _SC_ROWS // _TC_BLOCK

    def body(w_ref, sc_ref, o_ref):
        del sc_ref
        o_ref[...] = w_ref[...]

    return pl.pallas_call(
        body,
        grid=((_ROWS - _SC_ROWS) // _TC_BLOCK,),
        out_shape=jax.ShapeDtypeStruct((_ROWS, _DIM), jnp.float32),
        in_specs=[
            pl.BlockSpec((_TC_BLOCK, _DIM), lambda i: (i + base, 0)),
            pl.BlockSpec(memory_space=pl.ANY),
        ],
        out_specs=pl.BlockSpec((_TC_BLOCK, _DIM), lambda i: (i + base, 0)),
        input_output_aliases={1: 0},
    )(weights, sc_out)


def kernel(input_ids, weights):
    del input_ids
    return _copy_rows_tc(weights, _gather_rows_sc(weights))


# speedup vs baseline: 1.0371x; 1.0371x over previous
import functools

import jax
import jax.numpy as jnp
from jax import lax
from jax.experimental import pallas as pl
from jax.experimental.pallas import tpu as pltpu
from jax.experimental.pallas import tpu_sc as plsc

_ROWS = 8192
_DIM = 1024
_NUM_WORKERS = 32
_CHUNK = 16
_NUM_CHUNKS = _ROWS // (_NUM_WORKERS * _CHUNK)
_NBUF = 6
_DELAY = 3

_MESH = plsc.VectorSubcoreMesh(core_axis_name="c", subcore_axis_name="s")


@functools.partial(
    pl.kernel,
    mesh=_MESH,
    out_type=jax.ShapeDtypeStruct((_ROWS, _DIM), jnp.float32),
    scratch_types=(
        [pltpu.VMEM((_CHUNK, _DIM), jnp.float32) for _ in range(_NBUF)]
        + [pltpu.SemaphoreType.DMA for _ in range(2 * _NBUF)]
    ),
)
def _gather_rows_sc(table_hbm, out_hbm, *scratch):
    bufs = scratch[:_NBUF]
    lsems = scratch[_NBUF : 2 * _NBUF]
    ssems = scratch[2 * _NBUF :]

    wid = lax.axis_index("s") * 2 + lax.axis_index("c")

    loads = [None] * _NBUF
    stores = [None] * _NBUF

    def _row0(j):
        return (j * _NUM_WORKERS + wid) * _CHUNK

    def _load(j):
        b = j % _NBUF
        loads[b] = pltpu.async_copy(
            table_hbm.at[pl.ds(_row0(j), _CHUNK)], bufs[b], lsems[b]
        )

    for j in range(min(_DELAY + 1, _NUM_CHUNKS)):
        _load(j)
    for i in range(_NUM_CHUNKS):
        b = i % _NBUF
        loads[b].wait()
        stores[b] = pltpu.async_copy(
            bufs[b], out_hbm.at[pl.ds(_row0(i), _CHUNK)], ssems[b]
        )
        j = i + _DELAY
        if _DELAY < j < _NUM_CHUNKS:
            bb = j % _NBUF
            if stores[bb] is not None:
                stores[bb].wait()
                stores[bb] = None
            _load(j)
    for st in stores:
        if st is not None:
            st.wait()


def kernel(input_ids, weights):
    del input_ids
    return _gather_rows_sc(weights)
